# baseline (speedup 1.0000x reference)
import functools

import jax
import jax.numpy as jnp
import numpy as np
from jax import lax
from jax.experimental import pallas as pl
from jax.experimental.pallas import tpu as pltpu
from jax.experimental.pallas import tpu_sc as plsc

N_CP = 20
L = 16
NC, NS = 2, 16
NW = NC * NS
WC = 131072
WC_SHIFT = WC.bit_length() - 1

_S_VALS = tuple(float(v) for v in np.linspace(0.0, 1.0, N_CP + 1)[1:].astype(np.float32))


@functools.lru_cache(maxsize=None)
def _make_tc_flatten(n_items: int):
  n_chunks = -(-n_items // WC)
  n_jt = -(-N_CP // 8)

  def body(in_ref, out_ref):
    for jr in range(8):
      out_ref[pl.ds(jr * WC, WC)] = in_ref[jr, :]

  return pl.pallas_call(
      body,
      grid=(n_jt, n_chunks),
      in_specs=[pl.BlockSpec((8, WC), lambda a, c: (a, c))],
      out_specs=pl.BlockSpec((8 * WC,), lambda a, c: (a * n_chunks + c,)),
      out_shape=jax.ShapeDtypeStruct((n_jt * n_chunks * 8 * WC,), jnp.float32),
  )


@functools.lru_cache(maxsize=None)
def _make_sc_kernel(B: int, n_items: int):
  b_per_w = B // NW
  n_chunks = b_per_w // L
  mesh = plsc.VectorSubcoreMesh(
      core_axis_name="c", subcore_axis_name="s", num_cores=NC, num_subcores=NS)

  @functools.partial(
      pl.kernel,
      mesh=mesh,
      compiler_params=pltpu.CompilerParams(
          needs_layout_passes=False, use_tc_tiling_on_sc=False),
      out_type=jax.ShapeDtypeStruct((B,), jnp.float32),
      scratch_types=[
          pltpu.VMEM((b_per_w,), jnp.int32),
          pltpu.VMEM((b_per_w,), jnp.float32),
          pltpu.VMEM((b_per_w,), jnp.float32),
          pltpu.VMEM((b_per_w,), jnp.float32),
          pltpu.VMEM((N_CP * b_per_w,), jnp.int32),
          pltpu.VMEM((N_CP * b_per_w,), jnp.float32),
          pltpu.VMEM((b_per_w,), jnp.float32),
          pltpu.SemaphoreType.DMA,
          pltpu.SemaphoreType.DMA,
          pltpu.SemaphoreType.DMA,
      ],
  )
  def trend_kernel(t_hbm, ids_hbm, m_hbm, k_hbm, d_hbm, out_hbm,
                   idx_v, t_v, m_v, k_v, idx2_v, d_v, o_v, sem_m, sem_k, sem_d):
    wid = lax.axis_index("s") * NC + lax.axis_index("c")
    base = wid * b_per_w
    pltpu.sync_copy(ids_hbm.at[pl.ds(base, b_per_w)], idx_v)
    cm = pltpu.async_copy(m_hbm.at[idx_v], m_v, sem_m)
    ck = pltpu.async_copy(k_hbm.at[idx_v], k_v, sem_k)
    pltpu.sync_copy(t_hbm.at[pl.ds(base, b_per_w)], t_v)

    n_cflat = -(-n_items // WC)

    def build_body(c, _):
      off = c * L
      ids_vec = idx_v[pl.ds(off, L)]
      flat0 = (ids_vec >> WC_SHIFT) * (8 * WC) + (ids_vec & (WC - 1))
      for j in range(N_CP):
        a, jr = divmod(j, 8)
        idx2_v[pl.ds(j * b_per_w + off, L)] = (
            flat0 + (a * n_cflat * 8 + jr) * WC)
      return 0

    lax.fori_loop(0, n_chunks, build_body, 0)
    cd = pltpu.async_copy(d_hbm.at[idx2_v], d_v, sem_d)
    cd.wait()
    cm.wait()
    ck.wait()

    def chunk_body(c, _):
      off = c * L
      tt = t_v[pl.ds(off, L)]
      acc = tt * k_v[pl.ds(off, L)] + m_v[pl.ds(off, L)]
      for j in range(N_CP):
        sj = _S_VALS[j]
        acc = acc + jnp.where(tt > sj, tt - sj, 0.0) * d_v[pl.ds(j * b_per_w + off, L)]
      o_v[pl.ds(off, L)] = acc
      return 0

    lax.fori_loop(0, n_chunks, chunk_body, 0)
    pltpu.sync_copy(o_v, out_hbm.at[pl.ds(base, b_per_w)])

  return trend_kernel


def kernel(t, ids, emb_m, emb_k, emb_delta):
  B = t.shape[0]
  n_items = emb_delta.shape[0]
  dflat = _make_tc_flatten(n_items)(emb_delta.T)
  out = _make_sc_kernel(B, n_items)(
      t.reshape(B), ids.reshape(B), emb_m.reshape(n_items),
      emb_k.reshape(n_items), dflat)
  return out.reshape(B, 1)


# speedup vs baseline: 1.0984x; 1.0984x over previous
import functools

import jax
import jax.numpy as jnp
import numpy as np
from jax import lax
from jax.experimental import pallas as pl
from jax.experimental.pallas import tpu as pltpu
from jax.experimental.pallas import tpu_sc as plsc

N_CP = 20
L = 16
NC, NS = 2, 16
NW = NC * NS
WC = 65536
SC_LO, SC_HI = 5, 15
N_B = SC_HI - SC_LO
WP = 8192

_S_VALS = tuple(float(v) for v in np.linspace(0.0, 1.0, N_CP + 1)[1:].astype(np.float32))


@functools.lru_cache(maxsize=None)
def _make_tc_flatten(n_items: int):
  n_chunks = -(-n_items // WC)
  chunks_a = list(range(SC_LO)) + list(range(SC_HI, n_chunks))
  n_a = len(chunks_a)

  def body(in_ref, out_ref):
    for j in range(N_CP):
      out_ref[pl.ds(j * WC, WC)] = in_ref[j, :]

  def in_map(g):
    return (0, jnp.where(g < SC_LO, g, g - SC_LO + SC_HI))

  return pl.pallas_call(
      body,
      grid=(n_a,),
      in_specs=[pl.BlockSpec((N_CP, WC), in_map)],
      out_specs=pl.BlockSpec((N_CP * WC,), lambda g: (g,)),
      out_shape=jax.ShapeDtypeStruct((n_a * N_CP * WC,), jnp.float32),
  )


@functools.lru_cache(maxsize=None)
def _make_sc_flatten(n_items: int):
  pieces = WC // WP
  n_tasks = N_B * pieces * 3
  per_w = -(-n_tasks // NW)
  mesh = plsc.VectorSubcoreMesh(
      core_axis_name="c", subcore_axis_name="s", num_cores=NC, num_subcores=NS)

  @functools.partial(
      pl.kernel,
      mesh=mesh,
      compiler_params=pltpu.CompilerParams(
          needs_layout_passes=False, use_tc_tiling_on_sc=True),
      out_type=jax.ShapeDtypeStruct((N_B * N_CP * WC,), jnp.float32),
      scratch_types=[pltpu.VMEM((8, WP), jnp.float32)],
  )
  def sc_flatten(dt_hbm, out_hbm, buf_v):
    wid = lax.axis_index("s") * NC + lax.axis_index("c")
    for i in range(per_w):
      task = wid + i * NW
      cb = task // (pieces * 3)
      r = task % (pieces * 3)
      p, a = r // 3, r % 3
      col = pl.multiple_of((cb + SC_LO) * WC + p * WP, 128)

      def _move(nrows, row0):
        pltpu.sync_copy(dt_hbm.at[pl.ds(row0, nrows), pl.ds(col, WP)],
                        buf_v.at[pl.ds(0, nrows)])
        for jr in range(nrows):
          dst = cb * (N_CP * WC) + (row0 + jr) * WC + p * WP
          pltpu.sync_copy(buf_v.at[jr], out_hbm.at[pl.ds(dst, WP)])

      @pl.when(jnp.logical_and(task < n_tasks, a == 0))
      def _a0():
        _move(8, 0)

      @pl.when(jnp.logical_and(task < n_tasks, a == 1))
      def _a1():
        _move(8, 8)

      @pl.when(jnp.logical_and(task < n_tasks, a == 2))
      def _a2():
        _move(4, 16)

  return sc_flatten


@functools.lru_cache(maxsize=None)
def _make_sc_gather(B: int, n_items: int):
  b_per_w = B // NW
  n_chunks = b_per_w // L
  mesh = plsc.VectorSubcoreMesh(
      core_axis_name="c", subcore_axis_name="s", num_cores=NC, num_subcores=NS)

  @functools.partial(
      pl.kernel,
      mesh=mesh,
      compiler_params=pltpu.CompilerParams(
          needs_layout_passes=False, use_tc_tiling_on_sc=False),
      out_type=jax.ShapeDtypeStruct((B,), jnp.float32),
      scratch_types=[
          pltpu.VMEM((b_per_w,), jnp.int32),
          pltpu.VMEM((b_per_w,), jnp.float32),
          pltpu.VMEM((b_per_w,), jnp.float32),
          pltpu.VMEM((b_per_w,), jnp.float32),
          pltpu.VMEM((N_CP * b_per_w,), jnp.int32),
          pltpu.VMEM((N_CP * b_per_w,), jnp.int32),
          pltpu.VMEM((N_CP * b_per_w,), jnp.float32),
          pltpu.VMEM((N_CP * b_per_w,), jnp.float32),
          pltpu.VMEM((b_per_w,), jnp.float32),
          pltpu.SemaphoreType.DMA,
          pltpu.SemaphoreType.DMA,
          pltpu.SemaphoreType.DMA,
      ],
  )
  def trend_kernel(t_hbm, ids_hbm, m_hbm, k_hbm, da_hbm, db_hbm, out_hbm,
                   idx_v, t_v, m_v, k_v, ia_v, ib_v, da_v, db_v, o_v,
                   sem_m, sem_k, sem_d):
    wid = lax.axis_index("s") * NC + lax.axis_index("c")
    base = wid * b_per_w
    pltpu.sync_copy(ids_hbm.at[pl.ds(base, b_per_w)], idx_v)
    cm = pltpu.async_copy(m_hbm.at[idx_v], m_v, sem_m)
    ck = pltpu.async_copy(k_hbm.at[idx_v], k_v, sem_k)
    pltpu.sync_copy(t_hbm.at[pl.ds(base, b_per_w)], t_v)

    def build_body(c, _):
      off = c * L
      ids_vec = idx_v[pl.ds(off, L)]
      ch = ids_vec >> 16
      w = ids_vec & (WC - 1)
      pos_a = jnp.minimum(ch, SC_LO)
      pos_b = jnp.clip(ch - SC_LO, 0, N_B - 1)
      base_a = pos_a * (N_CP * WC) + w
      base_b = pos_b * (N_CP * WC) + w
      for j in range(N_CP):
        ia_v[pl.ds(j * b_per_w + off, L)] = base_a + j * WC
        ib_v[pl.ds(j * b_per_w + off, L)] = base_b + j * WC
      return 0

    lax.fori_loop(0, n_chunks, build_body, 0)
    ca = pltpu.async_copy(da_hbm.at[ia_v], da_v, sem_d)
    cb = pltpu.async_copy(db_hbm.at[ib_v], db_v, sem_d)
    ca.wait()
    cb.wait()
    cm.wait()
    ck.wait()

    def chunk_body(c, _):
      off = c * L
      tt = t_v[pl.ds(off, L)]
      ids_vec = idx_v[pl.ds(off, L)]
      sel_b = jnp.logical_and(ids_vec >= SC_LO * WC, ids_vec < SC_HI * WC)
      acc = tt * k_v[pl.ds(off, L)] + m_v[pl.ds(off, L)]
      for j in range(N_CP):
        sj = _S_VALS[j]
        dj = jnp.where(sel_b, db_v[pl.ds(j * b_per_w + off, L)],
                       da_v[pl.ds(j * b_per_w + off, L)])
        acc = acc + jnp.where(tt > sj, tt - sj, 0.0) * dj
      o_v[pl.ds(off, L)] = acc
      return 0

    lax.fori_loop(0, n_chunks, chunk_body, 0)
    pltpu.sync_copy(o_v, out_hbm.at[pl.ds(base, b_per_w)])

  return trend_kernel


def kernel(t, ids, emb_m, emb_k, emb_delta):
  B = t.shape[0]
  n_items = emb_delta.shape[0]
  dt = emb_delta.T
  dflat_a = _make_tc_flatten(n_items)(dt)
  dflat_b = _make_sc_flatten(n_items)(dt)
  out = _make_sc_gather(B, n_items)(
      t.reshape(B), ids.reshape(B), emb_m.reshape(n_items),
      emb_k.reshape(n_items), dflat_a, dflat_b)
  return out.reshape(B, 1)
